# baseline (speedup 1.0000x reference)
import functools

import jax
import jax.numpy as jnp
from jax import lax
from jax.experimental import pallas as pl
from jax.experimental.pallas import tpu as pltpu
from jax.experimental.pallas import tpu_sc as plsc

N = 10000
E = 320000
T = 12
CIN = 2
CT = 64
CS = 16
K = 3
G = 8

NC = 2
NS = 16
NW = NC * NS
CHK = 128
NCH = 80
BCH = 16
NBL = NCH // BCH
EPT = NCH * CHK
EP = NW * EPT
NCHT = 160
NBLT = NCHT // BCH
NPAD = 10240
NPT = NPAD // NS
W1 = CS * (T - 2)
W2 = CS * (T - 6)


def _mesh():
    return plsc.VectorSubcoreMesh(core_axis_name="c", subcore_axis_name="s")


def _make_mp(W, CHKm, BCHm, NBLTm, DEPTH):
    CW = W // 16
    SROWS = 128 if CHKm >= 128 else 80
    NSCP = NPT // SROWS

    @functools.partial(
        pl.kernel,
        out_type=jax.ShapeDtypeStruct((NC, NPAD, W), jnp.float32),
        mesh=_mesh(),
        compiler_params=pltpu.CompilerParams(use_tc_tiling_on_sc=False),
        scratch_types=[
            pltpu.VMEM((BCHm, CHKm), jnp.int32),
            pltpu.VMEM((BCHm, CHKm), jnp.int32),
            pltpu.VMEM((BCHm, CHKm), jnp.float32),
        ] + [pltpu.VMEM((CHKm, W), jnp.float32)] * DEPTH + [
            pltpu.VMEM_SHARED((NPAD, W), jnp.float32),
            pltpu.VMEM_SHARED((NPAD, W), jnp.float32),
        ] + [pltpu.SemaphoreType.DMA] * (2 * DEPTH),
    )
    def mp(tbl0_hbm, tbl1_hbm, src_hbm, dst_hbm, ew_hbm, out_hbm,
           src_v, dst_v, ew_v, *rest):
        rows = list(rest[:DEPTH])
        acc = rest[DEPTH]
        tblsp = rest[DEPTH + 1]
        gsem = list(rest[DEPTH + 2:2 * DEPTH + 2])
        ssem = list(rest[2 * DEPTH + 2:])
        r0, r1 = rows[0], rows[1]
        cid = lax.axis_index("c")
        sid = lax.axis_index("s")
        zero16 = jnp.zeros((16,), jnp.float32)

        def zrow(r, carry):
            for c in range(CW):
                r0[r, pl.ds(c * 16, 16)] = zero16
            return carry

        lax.fori_loop(0, SROWS, zrow, 0)
        base = sid * NPT
        for c5 in range(NSCP):
            pltpu.sync_copy(r0.at[pl.ds(0, SROWS)],
                            acc.at[pl.ds(base + c5 * SROWS, SROWS)])

        def stage_tbl(tbl):
            for c5 in range(NSCP):
                sl_n = pl.ds(base + c5 * SROWS, SROWS)
                pltpu.sync_copy(tbl.at[sl_n], r1.at[pl.ds(0, SROWS)])
                pltpu.sync_copy(r1.at[pl.ds(0, SROWS)], tblsp.at[sl_n])

        @pl.when(cid == 0)
        def _():
            stage_tbl(tbl0_hbm)

        @pl.when(cid == 1)
        def _():
            stage_tbl(tbl1_hbm)

        plsc.subcore_barrier()

        def scale(buf, jl):
            def group(g, c2):
                ewv = ew_v[jl, pl.ds(g * 16, 16)]
                for l in range(16):
                    w = ewv[l]
                    e = g * 16 + l
                    for c in range(CW):
                        sl = pl.ds(c * 16, 16)
                        buf[e, sl] = buf[e, sl] * w
                return c2

            lax.fori_loop(0, CHKm // 16, group, 0)

        def wait_gather(b):
            pltpu.make_async_copy(tbl0_hbm.at[pl.ds(0, CHKm)], rows[b],
                                  gsem[b]).wait()

        def wait_scatter(b):
            pltpu.make_async_copy(tbl0_hbm.at[pl.ds(0, CHKm)], rows[b],
                                  ssem[b]).wait()

        def block(blk, carry):
            @pl.when(blk > 0)
            def _():
                for b in range(DEPTH):
                    wait_scatter(b)

            sl_b = pl.ds(blk * BCHm, BCHm)
            pltpu.sync_copy(src_hbm.at[sid].at[sl_b], src_v)
            pltpu.sync_copy(dst_hbm.at[sid].at[sl_b], dst_v)
            pltpu.sync_copy(ew_hbm.at[sid].at[sl_b], ew_v)

            for b in range(DEPTH - 1):
                pltpu.async_copy(tblsp.at[src_v.at[b]], rows[b], gsem[b])

            for jl in range(BCHm):
                b = jl % DEPTH
                wait_gather(b)
                scale(rows[b], jl)
                if jl + DEPTH - 1 < BCHm:
                    bn = (jl + DEPTH - 1) % DEPTH
                    if jl >= 1:
                        wait_scatter(bn)
                    pltpu.async_copy(tblsp.at[src_v.at[jl + DEPTH - 1]],
                                     rows[bn], gsem[bn])
                pltpu.async_copy(rows[b], acc.at[dst_v.at[jl]], ssem[b],
                                 add=True)
            return carry

        lax.fori_loop(0, NBLTm, block, 0)
        for b in range(DEPTH):
            wait_scatter(b)

        plsc.subcore_barrier()

        for c5 in range(NSCP):
            sl_n = pl.ds(base + c5 * SROWS, SROWS)
            pltpu.sync_copy(acc.at[sl_n], r0.at[pl.ds(0, SROWS)])
            pltpu.sync_copy(r0.at[pl.ds(0, SROWS)], out_hbm.at[cid].at[sl_n])

    return mp


@functools.partial(
    pl.kernel,
    out_type=jax.ShapeDtypeStruct((NC, NPAD, 16), jnp.float32),
    mesh=_mesh(),
    compiler_params=pltpu.CompilerParams(use_tc_tiling_on_sc=False),
    scratch_types=[
        pltpu.VMEM((BCH, CHK), jnp.int32),
        pltpu.VMEM((BCH, CHK), jnp.float32),
        pltpu.VMEM((CHK, 16), jnp.float32),
        pltpu.VMEM_SHARED((NPAD, 16), jnp.float32),
    ],
)
def _deg_kernel(dst_hbm, ew_hbm, out_hbm, dst_v, ew_v, rows16, degacc):
    cid = lax.axis_index("c")
    sid = lax.axis_index("s")
    wid = cid * NS + sid
    zero16 = jnp.zeros((16,), jnp.float32)

    def zrow(r, carry):
        rows16[r, pl.ds(0, 16)] = zero16
        return carry

    lax.fori_loop(0, CHK, zrow, 0)
    base = sid * NPT
    for c5 in range(5):
        pltpu.sync_copy(rows16, degacc.at[pl.ds(base + c5 * CHK, CHK)])
    plsc.subcore_barrier()

    def blk(b, carry0):
        sl_b = pl.ds(b * BCH, BCH)
        pltpu.sync_copy(dst_hbm.at[wid].at[sl_b], dst_v)
        pltpu.sync_copy(ew_hbm.at[wid].at[sl_b], ew_v)

        def chunk(j, carry):
            def group(g, c2):
                ewv = ew_v[j, pl.ds(g * 16, 16)]
                for l in range(16):
                    rows16[g * 16 + l, pl.ds(0, 16)] = jnp.full(
                        (16,), ewv[l], jnp.float32)
                return c2

            lax.fori_loop(0, CHK // 16, group, 0)
            pltpu.sync_copy(rows16, degacc.at[dst_v.at[j]], add=True)
            return carry

        lax.fori_loop(0, BCH, chunk, 0)
        return carry0

    lax.fori_loop(0, NBL, blk, 0)
    plsc.subcore_barrier()

    for c5 in range(5):
        sl_n = pl.ds(base + c5 * CHK, CHK)
        pltpu.sync_copy(degacc.at[sl_n], rows16)
        pltpu.sync_copy(rows16, out_hbm.at[cid].at[sl_n])


NB1 = 1000


def _stage1_body(x2_ref, dega_ref, degb_ref, wbig_ref, bbig_ref, ws1_ref,
                 tlo_ref, thi_ref, dis_ref):
    deg = dega_ref[:, 0:1] + degb_ref[:, 0:1] + 1.0
    dis = lax.rsqrt(deg)
    dis_ref[...] = jnp.broadcast_to(dis, (NB1, 8))
    yall = jnp.dot(x2_ref[...].astype(jnp.bfloat16), wbig_ref[...],
                   preferred_element_type=jnp.float32) + bbig_ref[0:1, :]
    for t in range(T - 2):
        y = yall[:, t * 2 * CT:(t + 1) * 2 * CT]
        glu = y[:, :CT] * jax.nn.sigmoid(y[:, CT:])
        h16 = jnp.dot(glu, ws1_ref[...], preferred_element_type=jnp.float32)
        if t < 5:
            tlo_ref[:, t * CS:(t + 1) * CS] = h16 * dis
        else:
            thi_ref[:, (t - 5) * CS:(t - 4) * CS] = h16 * dis


def _stage1(x2, dega, degb, wbig, bbig, ws1):
    return pl.pallas_call(
        _stage1_body,
        grid=(N // NB1,),
        in_specs=[
            pl.BlockSpec((NB1, CIN * T), lambda i: (i, 0)),
            pl.BlockSpec((NB1, 16), lambda i: (i, 0)),
            pl.BlockSpec((NB1, 16), lambda i: (i, 0)),
            pl.BlockSpec((CIN * T, (T - 2) * 2 * CT), lambda i: (0, 0)),
            pl.BlockSpec((1, (T - 2) * 2 * CT), lambda i: (0, 0)),
            pl.BlockSpec((CT, CS), lambda i: (0, 0)),
        ],
        out_specs=[
            pl.BlockSpec((NB1, W1 // 2), lambda i: (i, 0)),
            pl.BlockSpec((NB1, W1 // 2), lambda i: (i, 0)),
            pl.BlockSpec((NB1, 8), lambda i: (i, 0)),
        ],
        out_shape=[
            jax.ShapeDtypeStruct((NPAD, W1 // 2), jnp.float32),
            jax.ShapeDtypeStruct((NPAD, W1 // 2), jnp.float32),
            jax.ShapeDtypeStruct((N, 8), jnp.float32),
        ],
    )(x2, dega, degb, wbig, bbig, ws1)


def _stage3_body(acca_ref, accb_ref, tlo_ref, thi_ref, dis_ref, w48_ref,
                 b1b_ref, w192_ref, b2a_ref, ws2_ref, brow_ref, olo_ref,
                 ohi_ref):
    dis = dis_ref[:, 0:1]
    z = jnp.concatenate([dis * (acca_ref[...] + tlo_ref[...]),
                         dis * (accb_ref[...] + thi_ref[...])], axis=1)
    z = jnp.maximum(z + brow_ref[0:1, :], 0.0).astype(jnp.bfloat16)
    glus = []
    for t in range(T - 4):
        y = jnp.dot(z[:, CS * t:CS * t + 3 * CS], w48_ref[...],
                    preferred_element_type=jnp.float32) + b1b_ref[0:1, :]
        glus.append((y[:, :CT] * jax.nn.sigmoid(y[:, CT:])).astype(jnp.bfloat16))
    z2 = jnp.concatenate(glus, axis=1)
    for t in range(T - 6):
        y = jnp.dot(z2[:, CT * t:CT * t + 3 * CT], w192_ref[...],
                    preferred_element_type=jnp.float32) + b2a_ref[0:1, :]
        glu = y[:, :CT] * jax.nn.sigmoid(y[:, CT:])
        h16 = jnp.dot(glu, ws2_ref[...], preferred_element_type=jnp.float32)
        if t < 3:
            olo_ref[:, t * CS:(t + 1) * CS] = h16 * dis
        else:
            ohi_ref[:, (t - 3) * CS:(t - 2) * CS] = h16 * dis


def _stage3(acca, accb, tlo, thi, dis8, w48, b1b, w192, b2a, ws2, brow160):
    return pl.pallas_call(
        _stage3_body,
        grid=(N // NB1,),
        in_specs=[
            pl.BlockSpec((NB1, W1 // 2), lambda i: (i, 0)),
            pl.BlockSpec((NB1, W1 // 2), lambda i: (i, 0)),
            pl.BlockSpec((NB1, W1 // 2), lambda i: (i, 0)),
            pl.BlockSpec((NB1, W1 // 2), lambda i: (i, 0)),
            pl.BlockSpec((NB1, 8), lambda i: (i, 0)),
            pl.BlockSpec((3 * CS, 2 * CT), lambda i: (0, 0)),
            pl.BlockSpec((1, 2 * CT), lambda i: (0, 0)),
            pl.BlockSpec((3 * CT, 2 * CT), lambda i: (0, 0)),
            pl.BlockSpec((1, 2 * CT), lambda i: (0, 0)),
            pl.BlockSpec((CT, CS), lambda i: (0, 0)),
            pl.BlockSpec((1, W1), lambda i: (0, 0)),
        ],
        out_specs=[
            pl.BlockSpec((NB1, W2 // 2), lambda i: (i, 0)),
            pl.BlockSpec((NB1, W2 // 2), lambda i: (i, 0)),
        ],
        out_shape=[
            jax.ShapeDtypeStruct((NPAD, W2 // 2), jnp.float32),
            jax.ShapeDtypeStruct((NPAD, W2 // 2), jnp.float32),
        ],
    )(acca, accb, tlo, thi, dis8, w48, b1b, w192, b2a, ws2, brow160)


NB5 = 2000


def _stage5a_body(acca_ref, accb_ref, tlo_ref, thi_ref, dis_ref, brow_ref,
                  batch_ref, sums_ref, cnt_ref):
    i = pl.program_id(0)
    dis = dis_ref[:, 0:1]
    z = jnp.concatenate([dis * (acca_ref[...] + tlo_ref[...]),
                         dis * (accb_ref[...] + thi_ref[...])], axis=1)
    z = jnp.maximum(z + brow_ref[0:1, :], 0.0)
    gi = lax.broadcasted_iota(jnp.int32, (G, NB5), 0)
    onehot = (gi == batch_ref[0]).astype(jnp.float32)
    part = jnp.dot(onehot, z, preferred_element_type=jnp.float32)
    cnt = jnp.sum(onehot, axis=1, keepdims=True)

    @pl.when(i == 0)
    def _():
        sums_ref[...] = jnp.zeros_like(sums_ref)
        cnt_ref[...] = jnp.zeros_like(cnt_ref)

    sums_ref[...] += part
    cnt_ref[...] += jnp.broadcast_to(cnt, (G, 8))


def _stage5a(acca, accb, tlo, thi, dis8, brow96, batch_row):
    return pl.pallas_call(
        _stage5a_body,
        grid=(N // NB5,),
        in_specs=[
            pl.BlockSpec((NB5, W2 // 2), lambda i: (i, 0)),
            pl.BlockSpec((NB5, W2 // 2), lambda i: (i, 0)),
            pl.BlockSpec((NB5, W2 // 2), lambda i: (i, 0)),
            pl.BlockSpec((NB5, W2 // 2), lambda i: (i, 0)),
            pl.BlockSpec((NB5, 8), lambda i: (i, 0)),
            pl.BlockSpec((1, W2), lambda i: (0, 0)),
            pl.BlockSpec((1, 1, NB5), lambda i: (i, 0, 0)),
        ],
        out_specs=[
            pl.BlockSpec((G, W2), lambda i: (0, 0)),
            pl.BlockSpec((G, 8), lambda i: (0, 0)),
        ],
        out_shape=[
            jax.ShapeDtypeStruct((G, W2), jnp.float32),
            jax.ShapeDtypeStruct((G, 8), jnp.float32),
        ],
    )(acca, accb, tlo, thi, dis8, brow96, batch_row)


def _stage5b_body(sums_ref, cnt_ref, w48b_ref, b2b_ref, out_ref):
    mean = sums_ref[...] / jnp.maximum(cnt_ref[:, 0:1], 1.0)
    for t in range(T - 8):
        y = jnp.dot(mean[:, CS * t:CS * t + 3 * CS], w48b_ref[...],
                    preferred_element_type=jnp.float32) + b2b_ref[0:1, :]
        out_ref[t] = y[:, :CT] * jax.nn.sigmoid(y[:, CT:])


def _stage5b(sums, cnt, w48b, b2b):
    return pl.pallas_call(
        _stage5b_body,
        out_shape=jax.ShapeDtypeStruct((T - 8, G, CT), jnp.float32),
    )(sums, cnt, w48b, b2b)


CHKA, BCHA, NBLTA = 128, 16, 10
_mp80 = _make_mp(W1 // 2, CHKA, BCHA, NBLTA, 2)
_mp48 = _make_mp(W2 // 2, CHKA, BCHA, NBLTA, 3)


def kernel(x, edge_index, edge_attr, batch, Wt1a, bt1a, Ws1, bs1, Wt1b, bt1b,
           Wt2a, bt2a, Ws2, bs2, Wt2b, bt2b):
    src = edge_index[0]
    dst = edge_index[1]
    pad = EP - E
    def padres(a, zero, ept, nch, chk):
        padn = NS * ept - E
        return jnp.concatenate([a, jnp.full((padn,), zero, a.dtype)]).reshape(
            NS, nch, chk)

    srcp1 = padres(src, 0, CHKA * NBLTA * BCHA, NBLTA * BCHA, CHKA)
    dstp1 = padres(dst, 0, CHKA * NBLTA * BCHA, NBLTA * BCHA, CHKA)
    ewp1 = padres(edge_attr, 0.0, CHKA * NBLTA * BCHA, NBLTA * BCHA, CHKA)
    srcp2, dstp2, ewp2 = srcp1, dstp1, ewp1

    pad = EP - E
    dstf = jnp.concatenate([dst, jnp.zeros((pad,), jnp.int32)])
    ewf = jnp.concatenate([edge_attr, jnp.zeros((pad,), jnp.float32)])
    degout = _deg_kernel(dstf.reshape(NW, NCH, CHK), ewf.reshape(NW, NCH, CHK))

    x2 = x.reshape(N, CIN * T)
    w6 = jnp.transpose(Wt1a, (1, 2, 0))
    wbig = jnp.zeros((CIN * T, (T - 2) * 2 * CT), jnp.float32)
    for t in range(T - 2):
        for k in range(K):
            wbig = wbig.at[:, t * 2 * CT:(t + 1) * 2 * CT].add(
                jnp.zeros((CIN, T, 2 * CT), jnp.float32)
                .at[:, t + k, :].set(w6[:, k, :]).reshape(CIN * T, 2 * CT))
    bbig = jnp.tile(bt1a, T - 2).reshape(1, (T - 2) * 2 * CT)
    t1lo, t1hi, dis8 = _stage1(x2, degout[0], degout[1],
                               wbig.astype(jnp.bfloat16), bbig, Ws1)

    acc1 = _mp80(t1lo, t1hi, srcp1, dstp1, ewp1)

    w48 = jnp.transpose(Wt1b, (2, 1, 0)).reshape(3 * CS, 2 * CT).astype(jnp.bfloat16)
    w192 = jnp.transpose(Wt2a, (2, 1, 0)).reshape(3 * CT, 2 * CT).astype(jnp.bfloat16)
    brow160 = jnp.tile(bs1, T - 2).reshape(1, W1)
    t2lo, t2hi = _stage3(acc1[0], acc1[1], t1lo, t1hi, dis8, w48,
                         bt1b.reshape(1, 2 * CT), w192, bt2a.reshape(1, 2 * CT),
                         Ws2, brow160)

    acc2 = _mp48(t2lo, t2hi, srcp2, dstp2, ewp2)

    brow96 = jnp.tile(bs2, T - 6).reshape(1, W2)
    sums, cnt = _stage5a(acc2[0], acc2[1], t2lo, t2hi, dis8, brow96,
                         batch.reshape(N // NB5, 1, NB5))

    w48b = jnp.transpose(Wt2b, (2, 1, 0)).reshape(3 * CS, 2 * CT)
    o = _stage5b(sums, cnt, w48b, bt2b.reshape(1, 2 * CT))
    return jnp.transpose(o, (1, 2, 0))


# speedup vs baseline: 1.0024x; 1.0024x over previous
import functools

import jax
import jax.numpy as jnp
from jax import lax
from jax.experimental import pallas as pl
from jax.experimental.pallas import tpu as pltpu
from jax.experimental.pallas import tpu_sc as plsc

N = 10000
E = 320000
T = 12
CIN = 2
CT = 64
CS = 16
K = 3
G = 8

NC = 2
NS = 16
NW = NC * NS
CHK = 128
NCH = 80
BCH = 16
NBL = NCH // BCH
EPT = NCH * CHK
EP = NW * EPT
NCHT = 160
NBLT = NCHT // BCH
NPAD = 10240
NPT = NPAD // NS
W1 = CS * (T - 2)
W2 = CS * (T - 6)


def _mesh():
    return plsc.VectorSubcoreMesh(core_axis_name="c", subcore_axis_name="s")


def _make_mp(W, CHKm, BCHm, NBLTm, DEPTH):
    CW = W // 16
    SROWS = 128 if CHKm >= 128 else 80
    NSCP = NPT // SROWS

    @functools.partial(
        pl.kernel,
        out_type=jax.ShapeDtypeStruct((NC, NPAD, W), jnp.float32),
        mesh=_mesh(),
        compiler_params=pltpu.CompilerParams(use_tc_tiling_on_sc=False),
        scratch_types=[
            pltpu.VMEM((BCHm, CHKm), jnp.int32),
            pltpu.VMEM((BCHm, CHKm), jnp.int32),
            pltpu.VMEM((BCHm, CHKm), jnp.float32),
        ] + [pltpu.VMEM((CHKm, W), jnp.float32)] * DEPTH + [
            pltpu.VMEM_SHARED((NPAD, W), jnp.float32),
            pltpu.VMEM_SHARED((NPAD, W), jnp.float32),
        ] + [pltpu.SemaphoreType.DMA] * (2 * DEPTH),
    )
    def mp(tbl0_hbm, tbl1_hbm, src_hbm, dst_hbm, ew_hbm, out_hbm,
           src_v, dst_v, ew_v, *rest):
        rows = list(rest[:DEPTH])
        acc = rest[DEPTH]
        tblsp = rest[DEPTH + 1]
        gsem = list(rest[DEPTH + 2:2 * DEPTH + 2])
        ssem = list(rest[2 * DEPTH + 2:])
        r0, r1 = rows[0], rows[1]
        cid = lax.axis_index("c")
        sid = lax.axis_index("s")
        zero16 = jnp.zeros((16,), jnp.float32)

        def zrow(r, carry):
            for c in range(CW):
                r0[r, pl.ds(c * 16, 16)] = zero16
            return carry

        lax.fori_loop(0, SROWS, zrow, 0)
        base = sid * NPT
        for c5 in range(NSCP):
            pltpu.sync_copy(r0.at[pl.ds(0, SROWS)],
                            acc.at[pl.ds(base + c5 * SROWS, SROWS)])

        def stage_tbl(tbl):
            for c5 in range(NSCP):
                sl_n = pl.ds(base + c5 * SROWS, SROWS)
                pltpu.sync_copy(tbl.at[sl_n], r1.at[pl.ds(0, SROWS)])
                pltpu.sync_copy(r1.at[pl.ds(0, SROWS)], tblsp.at[sl_n])

        @pl.when(cid == 0)
        def _():
            stage_tbl(tbl0_hbm)

        @pl.when(cid == 1)
        def _():
            stage_tbl(tbl1_hbm)

        plsc.subcore_barrier()

        def scale(buf, jl):
            def group(g, c2):
                ewv = ew_v[jl, pl.ds(g * 16, 16)]
                for l in range(16):
                    w = ewv[l]
                    e = g * 16 + l
                    for c in range(CW):
                        sl = pl.ds(c * 16, 16)
                        buf[e, sl] = buf[e, sl] * w
                return c2

            lax.fori_loop(0, CHKm // 16, group, 0)

        def wait_gather(b):
            pltpu.make_async_copy(tbl0_hbm.at[pl.ds(0, CHKm)], rows[b],
                                  gsem[b]).wait()

        def wait_scatter(b):
            pltpu.make_async_copy(tbl0_hbm.at[pl.ds(0, CHKm)], rows[b],
                                  ssem[b]).wait()

        def block(blk, carry):
            @pl.when(blk > 0)
            def _():
                for b in range(DEPTH):
                    wait_scatter(b)

            sl_b = pl.ds(blk * BCHm, BCHm)
            pltpu.sync_copy(src_hbm.at[sid].at[sl_b], src_v)
            pltpu.sync_copy(dst_hbm.at[sid].at[sl_b], dst_v)
            pltpu.sync_copy(ew_hbm.at[sid].at[sl_b], ew_v)

            for b in range(DEPTH - 1):
                pltpu.async_copy(tblsp.at[src_v.at[b]], rows[b], gsem[b])

            for jl in range(BCHm):
                b = jl % DEPTH
                wait_gather(b)
                scale(rows[b], jl)
                if jl + DEPTH - 1 < BCHm:
                    bn = (jl + DEPTH - 1) % DEPTH
                    if jl >= 1:
                        wait_scatter(bn)
                    pltpu.async_copy(tblsp.at[src_v.at[jl + DEPTH - 1]],
                                     rows[bn], gsem[bn])
                pltpu.async_copy(rows[b], acc.at[dst_v.at[jl]], ssem[b],
                                 add=True)
            return carry

        lax.fori_loop(0, NBLTm, block, 0)
        for b in range(DEPTH):
            wait_scatter(b)

        plsc.subcore_barrier()

        for c5 in range(NSCP):
            sl_n = pl.ds(base + c5 * SROWS, SROWS)
            pltpu.sync_copy(acc.at[sl_n], r0.at[pl.ds(0, SROWS)])
            pltpu.sync_copy(r0.at[pl.ds(0, SROWS)], out_hbm.at[cid].at[sl_n])

    return mp


@functools.partial(
    pl.kernel,
    out_type=jax.ShapeDtypeStruct((NC, NPAD, 16), jnp.float32),
    mesh=_mesh(),
    compiler_params=pltpu.CompilerParams(use_tc_tiling_on_sc=False),
    scratch_types=[
        pltpu.VMEM((BCH, CHK), jnp.int32),
        pltpu.VMEM((BCH, CHK), jnp.float32),
        pltpu.VMEM((CHK, 16), jnp.float32),
        pltpu.VMEM_SHARED((NPAD, 16), jnp.float32),
    ],
)
def _deg_kernel(dst_hbm, ew_hbm, out_hbm, dst_v, ew_v, rows16, degacc):
    cid = lax.axis_index("c")
    sid = lax.axis_index("s")
    zero16 = jnp.zeros((16,), jnp.float32)

    def zrow(r, carry):
        rows16[r, pl.ds(0, 16)] = zero16
        return carry

    lax.fori_loop(0, CHK, zrow, 0)
    base = sid * NPT
    for c5 in range(5):
        pltpu.sync_copy(rows16, degacc.at[pl.ds(base + c5 * CHK, CHK)])
    plsc.subcore_barrier()

    def blk(b, carry0):
        sl_b = pl.ds((cid * NBL + b) * BCH, BCH)
        pltpu.sync_copy(dst_hbm.at[sid].at[sl_b], dst_v)
        pltpu.sync_copy(ew_hbm.at[sid].at[sl_b], ew_v)

        def chunk(j, carry):
            def group(g, c2):
                ewv = ew_v[j, pl.ds(g * 16, 16)]
                for l in range(16):
                    rows16[g * 16 + l, pl.ds(0, 16)] = jnp.full(
                        (16,), ewv[l], jnp.float32)
                return c2

            lax.fori_loop(0, CHK // 16, group, 0)
            pltpu.sync_copy(rows16, degacc.at[dst_v.at[j]], add=True)
            return carry

        lax.fori_loop(0, BCH, chunk, 0)
        return carry0

    lax.fori_loop(0, NBL, blk, 0)
    plsc.subcore_barrier()

    for c5 in range(5):
        sl_n = pl.ds(base + c5 * CHK, CHK)
        pltpu.sync_copy(degacc.at[sl_n], rows16)
        pltpu.sync_copy(rows16, out_hbm.at[cid].at[sl_n])


NB1 = 1000


def _stage1_body(x2_ref, dega_ref, degb_ref, wbig_ref, bbig_ref, ws1_ref,
                 tlo_ref, thi_ref, dis_ref):
    deg = dega_ref[:, 0:1] + degb_ref[:, 0:1] + 1.0
    dis = lax.rsqrt(deg)
    dis_ref[...] = jnp.broadcast_to(dis, (NB1, 8))
    yall = jnp.dot(x2_ref[...].astype(jnp.bfloat16), wbig_ref[...],
                   preferred_element_type=jnp.float32) + bbig_ref[0:1, :]
    for t in range(T - 2):
        y = yall[:, t * 2 * CT:(t + 1) * 2 * CT]
        glu = y[:, :CT] * jax.nn.sigmoid(y[:, CT:])
        h16 = jnp.dot(glu, ws1_ref[...], preferred_element_type=jnp.float32)
        if t < 5:
            tlo_ref[:, t * CS:(t + 1) * CS] = h16 * dis
        else:
            thi_ref[:, (t - 5) * CS:(t - 4) * CS] = h16 * dis


def _stage1(x2, dega, degb, wbig, bbig, ws1):
    return pl.pallas_call(
        _stage1_body,
        grid=(N // NB1,),
        in_specs=[
            pl.BlockSpec((NB1, CIN * T), lambda i: (i, 0)),
            pl.BlockSpec((NB1, 16), lambda i: (i, 0)),
            pl.BlockSpec((NB1, 16), lambda i: (i, 0)),
            pl.BlockSpec((CIN * T, (T - 2) * 2 * CT), lambda i: (0, 0)),
            pl.BlockSpec((1, (T - 2) * 2 * CT), lambda i: (0, 0)),
            pl.BlockSpec((CT, CS), lambda i: (0, 0)),
        ],
        out_specs=[
            pl.BlockSpec((NB1, W1 // 2), lambda i: (i, 0)),
            pl.BlockSpec((NB1, W1 // 2), lambda i: (i, 0)),
            pl.BlockSpec((NB1, 8), lambda i: (i, 0)),
        ],
        out_shape=[
            jax.ShapeDtypeStruct((NPAD, W1 // 2), jnp.float32),
            jax.ShapeDtypeStruct((NPAD, W1 // 2), jnp.float32),
            jax.ShapeDtypeStruct((N, 8), jnp.float32),
        ],
    )(x2, dega, degb, wbig, bbig, ws1)


def _stage3_body(acca_ref, accb_ref, tlo_ref, thi_ref, dis_ref, w48_ref,
                 b1b_ref, w192_ref, b2a_ref, ws2_ref, brow_ref, olo_ref,
                 ohi_ref):
    dis = dis_ref[:, 0:1]
    z = jnp.concatenate([dis * (acca_ref[...] + tlo_ref[...]),
                         dis * (accb_ref[...] + thi_ref[...])], axis=1)
    z = jnp.maximum(z + brow_ref[0:1, :], 0.0).astype(jnp.bfloat16)
    glus = []
    for t in range(T - 4):
        y = jnp.dot(z[:, CS * t:CS * t + 3 * CS], w48_ref[...],
                    preferred_element_type=jnp.float32) + b1b_ref[0:1, :]
        glus.append((y[:, :CT] * jax.nn.sigmoid(y[:, CT:])).astype(jnp.bfloat16))
    z2 = jnp.concatenate(glus, axis=1)
    for t in range(T - 6):
        y = jnp.dot(z2[:, CT * t:CT * t + 3 * CT], w192_ref[...],
                    preferred_element_type=jnp.float32) + b2a_ref[0:1, :]
        glu = y[:, :CT] * jax.nn.sigmoid(y[:, CT:])
        h16 = jnp.dot(glu, ws2_ref[...], preferred_element_type=jnp.float32)
        if t < 3:
            olo_ref[:, t * CS:(t + 1) * CS] = h16 * dis
        else:
            ohi_ref[:, (t - 3) * CS:(t - 2) * CS] = h16 * dis


def _stage3(acca, accb, tlo, thi, dis8, w48, b1b, w192, b2a, ws2, brow160):
    return pl.pallas_call(
        _stage3_body,
        grid=(N // NB1,),
        in_specs=[
            pl.BlockSpec((NB1, W1 // 2), lambda i: (i, 0)),
            pl.BlockSpec((NB1, W1 // 2), lambda i: (i, 0)),
            pl.BlockSpec((NB1, W1 // 2), lambda i: (i, 0)),
            pl.BlockSpec((NB1, W1 // 2), lambda i: (i, 0)),
            pl.BlockSpec((NB1, 8), lambda i: (i, 0)),
            pl.BlockSpec((3 * CS, 2 * CT), lambda i: (0, 0)),
            pl.BlockSpec((1, 2 * CT), lambda i: (0, 0)),
            pl.BlockSpec((3 * CT, 2 * CT), lambda i: (0, 0)),
            pl.BlockSpec((1, 2 * CT), lambda i: (0, 0)),
            pl.BlockSpec((CT, CS), lambda i: (0, 0)),
            pl.BlockSpec((1, W1), lambda i: (0, 0)),
        ],
        out_specs=[
            pl.BlockSpec((NB1, W2 // 2), lambda i: (i, 0)),
            pl.BlockSpec((NB1, W2 // 2), lambda i: (i, 0)),
        ],
        out_shape=[
            jax.ShapeDtypeStruct((NPAD, W2 // 2), jnp.float32),
            jax.ShapeDtypeStruct((NPAD, W2 // 2), jnp.float32),
        ],
    )(acca, accb, tlo, thi, dis8, w48, b1b, w192, b2a, ws2, brow160)


NB5 = 2000


def _stage5a_body(acca_ref, accb_ref, tlo_ref, thi_ref, dis_ref, brow_ref,
                  batch_ref, sums_ref, cnt_ref):
    i = pl.program_id(0)
    dis = dis_ref[:, 0:1]
    z = jnp.concatenate([dis * (acca_ref[...] + tlo_ref[...]),
                         dis * (accb_ref[...] + thi_ref[...])], axis=1)
    z = jnp.maximum(z + brow_ref[0:1, :], 0.0)
    gi = lax.broadcasted_iota(jnp.int32, (G, NB5), 0)
    onehot = (gi == batch_ref[0]).astype(jnp.float32)
    part = jnp.dot(onehot, z, preferred_element_type=jnp.float32)
    cnt = jnp.sum(onehot, axis=1, keepdims=True)

    @pl.when(i == 0)
    def _():
        sums_ref[...] = jnp.zeros_like(sums_ref)
        cnt_ref[...] = jnp.zeros_like(cnt_ref)

    sums_ref[...] += part
    cnt_ref[...] += jnp.broadcast_to(cnt, (G, 8))


def _stage5a(acca, accb, tlo, thi, dis8, brow96, batch_row):
    return pl.pallas_call(
        _stage5a_body,
        grid=(N // NB5,),
        in_specs=[
            pl.BlockSpec((NB5, W2 // 2), lambda i: (i, 0)),
            pl.BlockSpec((NB5, W2 // 2), lambda i: (i, 0)),
            pl.BlockSpec((NB5, W2 // 2), lambda i: (i, 0)),
            pl.BlockSpec((NB5, W2 // 2), lambda i: (i, 0)),
            pl.BlockSpec((NB5, 8), lambda i: (i, 0)),
            pl.BlockSpec((1, W2), lambda i: (0, 0)),
            pl.BlockSpec((1, 1, NB5), lambda i: (i, 0, 0)),
        ],
        out_specs=[
            pl.BlockSpec((G, W2), lambda i: (0, 0)),
            pl.BlockSpec((G, 8), lambda i: (0, 0)),
        ],
        out_shape=[
            jax.ShapeDtypeStruct((G, W2), jnp.float32),
            jax.ShapeDtypeStruct((G, 8), jnp.float32),
        ],
    )(acca, accb, tlo, thi, dis8, brow96, batch_row)


def _stage5b_body(sums_ref, cnt_ref, w48b_ref, b2b_ref, out_ref):
    mean = sums_ref[...] / jnp.maximum(cnt_ref[:, 0:1], 1.0)
    for t in range(T - 8):
        y = jnp.dot(mean[:, CS * t:CS * t + 3 * CS], w48b_ref[...],
                    preferred_element_type=jnp.float32) + b2b_ref[0:1, :]
        out_ref[t] = y[:, :CT] * jax.nn.sigmoid(y[:, CT:])


def _stage5b(sums, cnt, w48b, b2b):
    return pl.pallas_call(
        _stage5b_body,
        out_shape=jax.ShapeDtypeStruct((T - 8, G, CT), jnp.float32),
    )(sums, cnt, w48b, b2b)


CHKA, BCHA, NBLTA = 128, 16, 10
_mp80 = _make_mp(W1 // 2, CHKA, BCHA, NBLTA, 2)
_mp48 = _make_mp(W2 // 2, CHKA, BCHA, NBLTA, 3)


def kernel(x, edge_index, edge_attr, batch, Wt1a, bt1a, Ws1, bs1, Wt1b, bt1b,
           Wt2a, bt2a, Ws2, bs2, Wt2b, bt2b):
    src = edge_index[0]
    dst = edge_index[1]
    pad = EP - E
    def padres(a, zero, ept, nch, chk):
        padn = NS * ept - E
        return jnp.concatenate([a, jnp.full((padn,), zero, a.dtype)]).reshape(
            NS, nch, chk)

    srcp1 = padres(src, 0, CHKA * NBLTA * BCHA, NBLTA * BCHA, CHKA)
    dstp1 = padres(dst, 0, CHKA * NBLTA * BCHA, NBLTA * BCHA, CHKA)
    ewp1 = padres(edge_attr, 0.0, CHKA * NBLTA * BCHA, NBLTA * BCHA, CHKA)
    srcp2, dstp2, ewp2 = srcp1, dstp1, ewp1

    degout = _deg_kernel(dstp1, ewp1)

    x2 = x.reshape(N, CIN * T)
    w6 = jnp.transpose(Wt1a, (1, 2, 0))
    wbig = jnp.zeros((CIN * T, (T - 2) * 2 * CT), jnp.float32)
    for t in range(T - 2):
        for k in range(K):
            wbig = wbig.at[:, t * 2 * CT:(t + 1) * 2 * CT].add(
                jnp.zeros((CIN, T, 2 * CT), jnp.float32)
                .at[:, t + k, :].set(w6[:, k, :]).reshape(CIN * T, 2 * CT))
    bbig = jnp.tile(bt1a, T - 2).reshape(1, (T - 2) * 2 * CT)
    t1lo, t1hi, dis8 = _stage1(x2, degout[0], degout[1],
                               wbig.astype(jnp.bfloat16), bbig, Ws1)

    acc1 = _mp80(t1lo, t1hi, srcp1, dstp1, ewp1)

    w48 = jnp.transpose(Wt1b, (2, 1, 0)).reshape(3 * CS, 2 * CT).astype(jnp.bfloat16)
    w192 = jnp.transpose(Wt2a, (2, 1, 0)).reshape(3 * CT, 2 * CT).astype(jnp.bfloat16)
    brow160 = jnp.tile(bs1, T - 2).reshape(1, W1)
    t2lo, t2hi = _stage3(acc1[0], acc1[1], t1lo, t1hi, dis8, w48,
                         bt1b.reshape(1, 2 * CT), w192, bt2a.reshape(1, 2 * CT),
                         Ws2, brow160)

    acc2 = _mp48(t2lo, t2hi, srcp2, dstp2, ewp2)

    brow96 = jnp.tile(bs2, T - 6).reshape(1, W2)
    sums, cnt = _stage5a(acc2[0], acc2[1], t2lo, t2hi, dis8, brow96,
                         batch.reshape(N // NB5, 1, NB5))

    w48b = jnp.transpose(Wt2b, (2, 1, 0)).reshape(3 * CS, 2 * CT)
    o = _stage5b(sums, cnt, w48b, bt2b.reshape(1, 2 * CT))
    return jnp.transpose(o, (1, 2, 0))
